# baseline (speedup 1.0000x reference)
import jax
import jax.numpy as jnp
from jax.experimental import pallas as pl


def _mlp_body(x_ref, eps_ref, w1_ref, b1_ref, w2_ref, b2_ref, o_ref):
    xe = jnp.concatenate([x_ref[...], eps_ref[...]], axis=1)
    h = jnp.dot(xe, w1_ref[...], preferred_element_type=jnp.float32,
                precision=jax.lax.Precision.DEFAULT)
    h = jnp.maximum(h + b1_ref[...], 0.0)
    o = jnp.dot(h, w2_ref[...], preferred_element_type=jnp.float32,
                precision=jax.lax.Precision.DEFAULT)
    o = o + b2_ref[...]
    o_ref[...] = jnp.maximum(o, 0.0) + jnp.log1p(jnp.exp(-jnp.abs(o)))


def kernel(x, eps, W1, b1, W2, b2):
    B, d_in = x.shape
    noise_dim = eps.shape[1]
    H = W1.shape[1]
    d_out = W2.shape[1]

    b1r = b1.reshape(1, H)
    b2r = b2.reshape(1, d_out)

    bm = 256
    grid = (B // bm,)

    return pl.pallas_call(
        _mlp_body,
        grid=grid,
        in_specs=[
            pl.BlockSpec((bm, d_in), lambda i: (i, 0)),
            pl.BlockSpec((bm, noise_dim), lambda i: (i, 0)),
            pl.BlockSpec((d_in + noise_dim, H), lambda i: (0, 0)),
            pl.BlockSpec((1, H), lambda i: (0, 0)),
            pl.BlockSpec((H, d_out), lambda i: (0, 0)),
            pl.BlockSpec((1, d_out), lambda i: (0, 0)),
        ],
        out_specs=pl.BlockSpec((bm, d_out), lambda i: (i, 0)),
        out_shape=jax.ShapeDtypeStruct((B, d_out), jnp.float32),
    )(x, eps, W1, b1r, W2, b2r)


# speedup vs baseline: 1.3611x; 1.3611x over previous
import jax
import jax.numpy as jnp
from jax.experimental import pallas as pl


def _mlp_body(x_ref, eps_ref, w1_ref, b1_ref, w2_ref, b2_ref, o_ref):
    xe = jnp.concatenate([x_ref[...], eps_ref[...]], axis=1)
    h = jnp.dot(xe, w1_ref[...], preferred_element_type=jnp.float32,
                precision=jax.lax.Precision.DEFAULT)
    h = jnp.maximum(h + b1_ref[...], 0.0)
    o = jnp.dot(h, w2_ref[...], preferred_element_type=jnp.float32,
                precision=jax.lax.Precision.DEFAULT)
    o = o + b2_ref[...]
    o_ref[...] = jnp.maximum(o, 0.0) + jnp.log1p(jnp.exp(-jnp.abs(o)))


def kernel(x, eps, W1, b1, W2, b2):
    B, d_in = x.shape
    noise_dim = eps.shape[1]
    H = W1.shape[1]
    d_out = W2.shape[1]

    b1r = b1.reshape(1, H)
    b2r = b2.reshape(1, d_out)

    bm = 1024
    grid = (B // bm,)

    return pl.pallas_call(
        _mlp_body,
        grid=grid,
        in_specs=[
            pl.BlockSpec((bm, d_in), lambda i: (i, 0)),
            pl.BlockSpec((bm, noise_dim), lambda i: (i, 0)),
            pl.BlockSpec((d_in + noise_dim, H), lambda i: (0, 0)),
            pl.BlockSpec((1, H), lambda i: (0, 0)),
            pl.BlockSpec((H, d_out), lambda i: (0, 0)),
            pl.BlockSpec((1, d_out), lambda i: (0, 0)),
        ],
        out_specs=pl.BlockSpec((bm, d_out), lambda i: (i, 0)),
        out_shape=jax.ShapeDtypeStruct((B, d_out), jnp.float32),
    )(x, eps, W1, b1r, W2, b2r)
